# Initial kernel scaffold; baseline (speedup 1.0000x reference)
import jax
import jax.numpy as jnp
from jax.experimental import pallas as pl


def kernel(encoded_cfg_nodes, edge_index, W1, b1, W2, b2, W3, b3):
    raise NotImplementedError("write your pallas kernel here")


# speedup vs baseline: 6.7946x; 6.7946x over previous
import functools

import jax
import jax.numpy as jnp
from jax import lax
from jax.experimental import pallas as pl
from jax.experimental.pallas import tpu as pltpu
from jax.experimental.pallas import tpu_sc as plsc

N = 10000
NPAD = 10112
D = 128
NC, NS = 2, 16
NBLK = 80
BLK = 128
EP = NC * NS * NBLK * BLK
RPT = NPAD // NS


def _sc_deg(dst4, ones16, zdeg):
  mesh = plsc.VectorSubcoreMesh(core_axis_name="c", subcore_axis_name="s")

  @functools.partial(
      pl.kernel,
      out_type=jax.ShapeDtypeStruct((NC, NPAD, D), jnp.float32),
      mesh=mesh,
      scratch_types=[
          pltpu.VMEM((NBLK, BLK), jnp.int32),
          pltpu.VMEM((BLK, D), jnp.float32),
          pltpu.VMEM_SHARED((NPAD, D), jnp.float32),
      ],
  )
  def k(dst_hbm, ones_hbm, z_hbm, out_hbm, dst_v, ones_v, acc):
    c = lax.axis_index("c")
    s = lax.axis_index("s")
    r0 = s * RPT
    pltpu.sync_copy(z_hbm, acc.at[pl.ds(r0, RPT)])
    pltpu.sync_copy(ones_hbm, ones_v)
    pltpu.sync_copy(dst_hbm.at[c, s], dst_v)
    plsc.subcore_barrier()

    def body(j, carry):
      pltpu.sync_copy(ones_v, acc.at[dst_v.at[j]], add=True)
      return carry

    lax.fori_loop(0, NBLK, body, 0)
    plsc.subcore_barrier()
    pltpu.sync_copy(acc.at[pl.ds(r0, RPT)], out_hbm.at[c, pl.ds(r0, RPT)])

  return k(dst4, ones16, zdeg)


def _sc_agg(g, src4, dst4, zbig):
  mesh = plsc.VectorSubcoreMesh(core_axis_name="c", subcore_axis_name="s")

  @functools.partial(
      pl.kernel,
      out_type=jax.ShapeDtypeStruct((NC, NPAD, D), jnp.float32),
      mesh=mesh,
      scratch_types=[
          pltpu.VMEM((NBLK, BLK), jnp.int32),
          pltpu.VMEM((NBLK, BLK), jnp.int32),
          pltpu.VMEM((BLK, D), jnp.float32),
          pltpu.VMEM_SHARED((NPAD, D), jnp.float32),
          pltpu.SemaphoreType.DMA,
      ],
  )
  def k(g_hbm, src_hbm, dst_hbm, z_hbm, out_hbm, src_v, dst_v, buf, acc, sem):
    c = lax.axis_index("c")
    s = lax.axis_index("s")
    r0 = s * RPT
    pltpu.sync_copy(z_hbm, acc.at[pl.ds(r0, RPT)])
    pltpu.sync_copy(src_hbm.at[c, s], src_v)
    pltpu.sync_copy(dst_hbm.at[c, s], dst_v)
    plsc.subcore_barrier()

    def body(j, carry):
      pltpu.async_copy(g_hbm.at[src_v.at[j]], buf, sem).wait()
      pltpu.sync_copy(buf, acc.at[dst_v.at[j]], add=True)
      return carry

    lax.fori_loop(0, NBLK, body, 0)
    plsc.subcore_barrier()
    pltpu.sync_copy(acc.at[pl.ds(r0, RPT)], out_hbm.at[c, pl.ds(r0, RPT)])

  return k(g, src4, dst4, zbig)


def _tc_pre(x, W, degp):
  def body(x_ref, w_ref, dp_ref, dinv_ref, g_ref):
    deg = dp_ref[0, :N, 0:1] + dp_ref[1, :N, 0:1] + 1.0
    dinv = lax.rsqrt(deg)
    dinv_ref[...] = dinv
    h = jnp.dot(x_ref[...], w_ref[...], preferred_element_type=jnp.float32)
    g_ref[...] = h * dinv

  return pl.pallas_call(
      body,
      out_shape=(
          jax.ShapeDtypeStruct((N, 1), jnp.float32),
          jax.ShapeDtypeStruct((N, D), jnp.float32),
      ),
  )(x, W, degp)


def _tc_mid(S, g, dinv, b, Wn):
  def body(s_ref, g_ref, dinv_ref, b_ref, w_ref, gout_ref):
    t = s_ref[0, :N, :] + s_ref[1, :N, :] + g_ref[...]
    xn = jnp.maximum(t * dinv_ref[...] + b_ref[...], 0.0)
    h = jnp.dot(xn, w_ref[...], preferred_element_type=jnp.float32)
    gout_ref[...] = h * dinv_ref[...]

  return pl.pallas_call(
      body,
      out_shape=jax.ShapeDtypeStruct((N, D), jnp.float32),
  )(S, g, dinv, b.reshape(1, D), Wn)


def _tc_fin(S, g, dinv, b):
  def body(s_ref, g_ref, dinv_ref, b_ref, out_ref):
    t = s_ref[0, :N, :] + s_ref[1, :N, :] + g_ref[...]
    out_ref[...] = jnp.maximum(t * dinv_ref[...] + b_ref[...], 0.0)

  return pl.pallas_call(
      body,
      out_shape=jax.ShapeDtypeStruct((N, D), jnp.float32),
  )(S, g, dinv, b.reshape(1, D))


@jax.jit
def kernel(encoded_cfg_nodes, edge_index, W1, b1, W2, b2, W3, b3):
  x = encoded_cfg_nodes
  src = edge_index[0].astype(jnp.int32)
  dst = edge_index[1].astype(jnp.int32)
  e = src.shape[0]
  pad = EP - e
  src4 = jnp.concatenate([src, jnp.zeros((pad,), jnp.int32)]).reshape(
      NC, NS, NBLK, BLK)
  dst4 = jnp.concatenate([dst, jnp.full((pad,), N, jnp.int32)]).reshape(
      NC, NS, NBLK, BLK)
  ones128 = jnp.ones((BLK, D), jnp.float32)
  zbig = jnp.zeros((RPT, D), jnp.float32)

  degp = _sc_deg(dst4, ones128, zbig)
  dinv, g = _tc_pre(x, W1, degp)
  S = _sc_agg(g, src4, dst4, zbig)
  g = _tc_mid(S, g, dinv, b1, W2)
  S = _sc_agg(g, src4, dst4, zbig)
  g = _tc_mid(S, g, dinv, b2, W3)
  S = _sc_agg(g, src4, dst4, zbig)
  return _tc_fin(S, g, dinv, b3)


# speedup vs baseline: 7.2705x; 1.0700x over previous
import functools

import jax
import jax.numpy as jnp
from jax import lax
from jax.experimental import pallas as pl
from jax.experimental.pallas import tpu as pltpu
from jax.experimental.pallas import tpu_sc as plsc

N = 10000
NPAD = 10112
D = 128
NC, NS = 2, 16
NBLK = 80
BLK = 128
EP = NC * NS * NBLK * BLK
NG = 10
KG = 8
RPT = NPAD // NS


def _sc_deg(dst4, ones16, zdeg):
  mesh = plsc.VectorSubcoreMesh(core_axis_name="c", subcore_axis_name="s")

  @functools.partial(
      pl.kernel,
      out_type=jax.ShapeDtypeStruct((NC, NPAD, D), jnp.float32),
      mesh=mesh,
      scratch_types=[
          pltpu.VMEM((NBLK, BLK), jnp.int32),
          pltpu.VMEM((BLK, D), jnp.float32),
          pltpu.VMEM_SHARED((NPAD, D), jnp.float32),
      ],
  )
  def k(dst_hbm, ones_hbm, z_hbm, out_hbm, dst_v, ones_v, acc):
    c = lax.axis_index("c")
    s = lax.axis_index("s")
    r0 = s * RPT
    pltpu.sync_copy(z_hbm, acc.at[pl.ds(r0, RPT)])
    pltpu.sync_copy(ones_hbm, ones_v)
    pltpu.sync_copy(dst_hbm.at[c, s], dst_v)
    plsc.subcore_barrier()

    def body(j, carry):
      pltpu.sync_copy(ones_v, acc.at[dst_v.at[j]], add=True)
      return carry

    lax.fori_loop(0, NBLK, body, 0)
    plsc.subcore_barrier()
    pltpu.sync_copy(acc.at[pl.ds(r0, RPT)], out_hbm.at[c, pl.ds(r0, RPT)])

  return k(dst4, ones16, zdeg)


def _sc_agg(g, idx6, zbig):
  mesh = plsc.VectorSubcoreMesh(core_axis_name="c", subcore_axis_name="s")

  @functools.partial(
      pl.kernel,
      out_type=jax.ShapeDtypeStruct((NC, NPAD, D), jnp.float32),
      mesh=mesh,
      scratch_types=[
          pltpu.VMEM((2, 2, KG, BLK), jnp.int32),
          pltpu.VMEM((2, BLK, D), jnp.float32),
          pltpu.VMEM_SHARED((NPAD, D), jnp.float32),
          pltpu.SemaphoreType.DMA((2,)),
          pltpu.SemaphoreType.DMA((2,)),
          pltpu.SemaphoreType.DMA((2,)),
      ],
  )
  def k(g_hbm, idx_hbm, z_hbm, out_hbm, idx_v, buf, acc, isem, gsem, ssem):
    c = lax.axis_index("c")
    s = lax.axis_index("s")
    r0 = s * RPT
    pltpu.sync_copy(z_hbm, acc.at[pl.ds(r0, RPT)])
    pltpu.async_copy(idx_hbm.at[c, s, 0], idx_v.at[0], isem.at[0])
    plsc.subcore_barrier()

    def body(grp, carry):
      pg = lax.rem(grp, 2)
      png = 1 - pg
      pltpu.make_async_copy(idx_hbm.at[c, s, grp], idx_v.at[pg],
                            isem.at[pg]).wait()

      @pl.when(grp < NG - 1)
      def _prefetch():
        pltpu.async_copy(idx_hbm.at[c, s, grp + 1], idx_v.at[png],
                         isem.at[png])

      gd = {}
      sd = {}
      gd[0] = pltpu.async_copy(g_hbm.at[idx_v.at[pg, 0, 0]], buf.at[0],
                               gsem.at[0])
      gd[1] = pltpu.async_copy(g_hbm.at[idx_v.at[pg, 0, 1]], buf.at[1],
                               gsem.at[1])
      for b in range(KG):
        bb = b % 2
        gd[b].wait()
        sd[b] = pltpu.async_copy(buf.at[bb], acc.at[idx_v.at[pg, 1, b]],
                                 ssem.at[bb], add=True)
        if b + 2 < KG:
          sd[b].wait()
          gd[b + 2] = pltpu.async_copy(g_hbm.at[idx_v.at[pg, 0, b + 2]],
                                       buf.at[bb], gsem.at[bb])
      sd[KG - 2].wait()
      sd[KG - 1].wait()
      return carry

    lax.fori_loop(0, NG, body, 0)
    plsc.subcore_barrier()
    pltpu.sync_copy(acc.at[pl.ds(r0, RPT)], out_hbm.at[c, pl.ds(r0, RPT)])

  return k(g, idx6, zbig)


def _tc_pre(x, W, degp):
  def body(x_ref, w_ref, dp_ref, dinv_ref, g_ref):
    deg = dp_ref[0, :N, 0:1] + dp_ref[1, :N, 0:1] + 1.0
    dinv = lax.rsqrt(deg)
    dinv_ref[...] = dinv
    h = jnp.dot(x_ref[...], w_ref[...], preferred_element_type=jnp.float32)
    g_ref[...] = h * dinv

  return pl.pallas_call(
      body,
      out_shape=(
          jax.ShapeDtypeStruct((N, 1), jnp.float32),
          jax.ShapeDtypeStruct((N, D), jnp.float32),
      ),
  )(x, W, degp)


def _tc_mid(S, g, dinv, b, Wn):
  def body(s_ref, g_ref, dinv_ref, b_ref, w_ref, gout_ref):
    t = s_ref[0, :N, :] + s_ref[1, :N, :] + g_ref[...]
    xn = jnp.maximum(t * dinv_ref[...] + b_ref[...], 0.0)
    h = jnp.dot(xn, w_ref[...], preferred_element_type=jnp.float32)
    gout_ref[...] = h * dinv_ref[...]

  return pl.pallas_call(
      body,
      out_shape=jax.ShapeDtypeStruct((N, D), jnp.float32),
  )(S, g, dinv, b.reshape(1, D), Wn)


def _tc_fin(S, g, dinv, b):
  def body(s_ref, g_ref, dinv_ref, b_ref, out_ref):
    t = s_ref[0, :N, :] + s_ref[1, :N, :] + g_ref[...]
    out_ref[...] = jnp.maximum(t * dinv_ref[...] + b_ref[...], 0.0)

  return pl.pallas_call(
      body,
      out_shape=jax.ShapeDtypeStruct((N, D), jnp.float32),
  )(S, g, dinv, b.reshape(1, D))


@jax.jit
def kernel(encoded_cfg_nodes, edge_index, W1, b1, W2, b2, W3, b3):
  x = encoded_cfg_nodes
  src = edge_index[0].astype(jnp.int32)
  dst = edge_index[1].astype(jnp.int32)
  e = src.shape[0]
  pad = EP - e
  srcp = jnp.concatenate([src, jnp.zeros((pad,), jnp.int32)]).reshape(
      NC, NS, NG, KG, BLK)
  dstp = jnp.concatenate([dst, jnp.full((pad,), N, jnp.int32)]).reshape(
      NC, NS, NG, KG, BLK)
  idx6 = jnp.stack([srcp, dstp], axis=3)
  dst4 = dstp.reshape(NC, NS, NBLK, BLK)
  ones128 = jnp.ones((BLK, D), jnp.float32)
  zbig = jnp.zeros((RPT, D), jnp.float32)

  degp = _sc_deg(dst4, ones128, zbig)
  dinv, g = _tc_pre(x, W1, degp)
  S = _sc_agg(g, idx6, zbig)
  g = _tc_mid(S, g, dinv, b1, W2)
  S = _sc_agg(g, idx6, zbig)
  g = _tc_mid(S, g, dinv, b2, W3)
  S = _sc_agg(g, idx6, zbig)
  return _tc_fin(S, g, dinv, b3)


# speedup vs baseline: 8.8756x; 1.2208x over previous
import functools

import jax
import jax.numpy as jnp
from jax import lax
from jax.experimental import pallas as pl
from jax.experimental.pallas import tpu as pltpu
from jax.experimental.pallas import tpu_sc as plsc

N = 10000
NPAD = 10112
D = 128
NC, NS = 2, 16
NBLK = 80
BLK = 128
EP = NC * NS * NBLK * BLK
KG = 8
NGT = 20
NG0 = 16
NG1 = NGT - NG0
RPT = NPAD // NS


def _sc_deg(dst4, ones16, zdeg):
  mesh = plsc.VectorSubcoreMesh(core_axis_name="c", subcore_axis_name="s")

  @functools.partial(
      pl.kernel,
      out_type=jax.ShapeDtypeStruct((NC, NPAD, D), jnp.float32),
      mesh=mesh,
      scratch_types=[
          pltpu.VMEM((NBLK, BLK), jnp.int32),
          pltpu.VMEM((BLK, D), jnp.float32),
          pltpu.VMEM_SHARED((NPAD, D), jnp.float32),
      ],
  )
  def k(dst_hbm, ones_hbm, z_hbm, out_hbm, dst_v, ones_v, acc):
    c = lax.axis_index("c")
    s = lax.axis_index("s")
    r0 = s * RPT
    pltpu.sync_copy(z_hbm, acc.at[pl.ds(r0, RPT)])
    pltpu.sync_copy(ones_hbm, ones_v)
    pltpu.sync_copy(dst_hbm.at[c, s], dst_v)
    plsc.subcore_barrier()

    def body(j, carry):
      pltpu.sync_copy(ones_v, acc.at[dst_v.at[j]], add=True)
      return carry

    lax.fori_loop(0, NBLK, body, 0)
    plsc.subcore_barrier()
    pltpu.sync_copy(acc.at[pl.ds(r0, RPT)], out_hbm.at[c, pl.ds(r0, RPT)])

  return k(dst4, ones16, zdeg)


def _sc_agg(g, idx6, zbig):
  mesh = plsc.VectorSubcoreMesh(core_axis_name="c", subcore_axis_name="s")

  @functools.partial(
      pl.kernel,
      out_type=jax.ShapeDtypeStruct((NC, NPAD, D), jnp.float32),
      mesh=mesh,
      scratch_types=[
          pltpu.VMEM((2, 2, KG, BLK), jnp.int32),
          pltpu.VMEM((2, BLK, D), jnp.float32),
          pltpu.VMEM_SHARED((NPAD, D), jnp.float32),
          pltpu.SemaphoreType.DMA((2,)),
          pltpu.SemaphoreType.DMA((2,)),
          pltpu.SemaphoreType.DMA((2,)),
      ],
  )
  def k(g_hbm, idx_hbm, z_hbm, out_hbm, idx_v, buf, acc, isem, gsem, ssem):
    c = lax.axis_index("c")
    s = lax.axis_index("s")
    r0 = s * RPT
    pltpu.sync_copy(z_hbm, acc.at[pl.ds(r0, RPT)])

    def run(base, ng):
      pltpu.async_copy(idx_hbm.at[s, base], idx_v.at[0], isem.at[0])

      def body(grp, carry):
        pg = lax.rem(grp, 2)
        png = 1 - pg
        pltpu.make_async_copy(idx_hbm.at[s, base + grp], idx_v.at[pg],
                              isem.at[pg]).wait()

        @pl.when(grp < ng - 1)
        def _prefetch():
          pltpu.async_copy(idx_hbm.at[s, base + grp + 1], idx_v.at[png],
                           isem.at[png])

        gd = {}
        sd = {}
        gd[0] = pltpu.async_copy(g_hbm.at[idx_v.at[pg, 0, 0]], buf.at[0],
                                 gsem.at[0])
        gd[1] = pltpu.async_copy(g_hbm.at[idx_v.at[pg, 0, 1]], buf.at[1],
                                 gsem.at[1])
        for b in range(KG):
          bb = b % 2
          gd[b].wait()
          sd[b] = pltpu.async_copy(buf.at[bb], acc.at[idx_v.at[pg, 1, b]],
                                   ssem.at[bb], add=True)
          if b + 2 < KG:
            sd[b].wait()
            gd[b + 2] = pltpu.async_copy(g_hbm.at[idx_v.at[pg, 0, b + 2]],
                                         buf.at[bb], gsem.at[bb])
        sd[KG - 2].wait()
        sd[KG - 1].wait()
        return carry

      lax.fori_loop(0, ng, body, 0)

    @pl.when(c == 0)
    def _c0():
      run(0, NG0)

    @pl.when(c == 1)
    def _c1():
      run(NG0, NG1)

    plsc.subcore_barrier()
    pltpu.sync_copy(acc.at[pl.ds(r0, RPT)], out_hbm.at[c, pl.ds(r0, RPT)])

  return k(g, idx6, zbig)


def _tc_pre(x, W, degp):
  def body(x_ref, w_ref, dp_ref, dinv_ref, g_ref):
    deg = dp_ref[0, :N, 0:1] + dp_ref[1, :N, 0:1] + 1.0
    dinv = lax.rsqrt(deg)
    dinv_ref[...] = dinv
    h = jnp.dot(x_ref[...], w_ref[...], preferred_element_type=jnp.float32)
    g_ref[...] = h * dinv

  return pl.pallas_call(
      body,
      out_shape=(
          jax.ShapeDtypeStruct((N, 1), jnp.float32),
          jax.ShapeDtypeStruct((N, D), jnp.float32),
      ),
  )(x, W, degp)


def _tc_mid(S, g, dinv, b, Wn):
  def body(s_ref, g_ref, dinv_ref, b_ref, w_ref, gout_ref):
    t = s_ref[0, :N, :] + s_ref[1, :N, :] + g_ref[...]
    xn = jnp.maximum(t * dinv_ref[...] + b_ref[...], 0.0)
    h = jnp.dot(xn, w_ref[...], preferred_element_type=jnp.float32)
    gout_ref[...] = h * dinv_ref[...]

  return pl.pallas_call(
      body,
      out_shape=jax.ShapeDtypeStruct((N, D), jnp.float32),
  )(S, g, dinv, b.reshape(1, D), Wn)


def _tc_fin(S, g, dinv, b):
  def body(s_ref, g_ref, dinv_ref, b_ref, out_ref):
    t = s_ref[0, :N, :] + s_ref[1, :N, :] + g_ref[...]
    out_ref[...] = jnp.maximum(t * dinv_ref[...] + b_ref[...], 0.0)

  return pl.pallas_call(
      body,
      out_shape=jax.ShapeDtypeStruct((N, D), jnp.float32),
  )(S, g, dinv, b.reshape(1, D))


@jax.jit
def kernel(encoded_cfg_nodes, edge_index, W1, b1, W2, b2, W3, b3):
  x = encoded_cfg_nodes
  src = edge_index[0].astype(jnp.int32)
  dst = edge_index[1].astype(jnp.int32)
  e = src.shape[0]
  pad = EP - e
  srcp = jnp.concatenate([src, jnp.zeros((pad,), jnp.int32)]).reshape(
      NS, NGT, KG, BLK)
  dstp = jnp.concatenate([dst, jnp.full((pad,), N, jnp.int32)]).reshape(
      NS, NGT, KG, BLK)
  idx6 = jnp.stack([srcp, dstp], axis=2)
  dst4 = dstp.reshape(NC, NS, NBLK, BLK)
  ones128 = jnp.ones((BLK, D), jnp.float32)
  zbig = jnp.zeros((RPT, D), jnp.float32)

  degp = _sc_deg(dst4, ones128, zbig)
  dinv, g = _tc_pre(x, W1, degp)
  S = _sc_agg(g, idx6, zbig)
  g = _tc_mid(S, g, dinv, b1, W2)
  S = _sc_agg(g, idx6, zbig)
  g = _tc_mid(S, g, dinv, b2, W3)
  S = _sc_agg(g, idx6, zbig)
  return _tc_fin(S, g, dinv, b3)


# speedup vs baseline: 9.3286x; 1.0510x over previous
import functools

import jax
import jax.numpy as jnp
from jax import lax
from jax.experimental import pallas as pl
from jax.experimental.pallas import tpu as pltpu
from jax.experimental.pallas import tpu_sc as plsc

N = 10000
NPAD = 10112
D = 128
NC, NS = 2, 16
NBLK = 80
BLK = 128
EP = NC * NS * NBLK * BLK
KG = 8
NGT = 20
NG0 = 15
NG1 = NGT - NG0
RPT = NPAD // NS


def _sc_deg(dst4, ones16, zdeg):
  mesh = plsc.VectorSubcoreMesh(core_axis_name="c", subcore_axis_name="s")

  @functools.partial(
      pl.kernel,
      out_type=jax.ShapeDtypeStruct((NC, NPAD, D), jnp.float32),
      mesh=mesh,
      scratch_types=[
          pltpu.VMEM((NBLK, BLK), jnp.int32),
          pltpu.VMEM((BLK, D), jnp.float32),
          pltpu.VMEM_SHARED((NPAD, D), jnp.float32),
          pltpu.SemaphoreType.DMA((8,)),
      ],
  )
  def k(dst_hbm, ones_hbm, z_hbm, out_hbm, dst_v, ones_v, acc, ssem):
    c = lax.axis_index("c")
    s = lax.axis_index("s")
    r0 = s * RPT
    pltpu.sync_copy(z_hbm, acc.at[pl.ds(r0, RPT)])
    pltpu.sync_copy(ones_hbm, ones_v)
    pltpu.sync_copy(dst_hbm.at[c, s], dst_v)
    plsc.subcore_barrier()

    def body(j, carry):
      sd = [
          pltpu.async_copy(ones_v, acc.at[dst_v.at[j * 8 + b]], ssem.at[b],
                           add=True) for b in range(8)
      ]
      for d in sd:
        d.wait()
      return carry

    lax.fori_loop(0, NBLK // 8, body, 0)
    plsc.subcore_barrier()
    pltpu.sync_copy(acc.at[pl.ds(r0, RPT)], out_hbm.at[c, pl.ds(r0, RPT)])

  return k(dst4, ones16, zdeg)


def _sc_agg(g, idx6, zbig):
  mesh = plsc.VectorSubcoreMesh(core_axis_name="c", subcore_axis_name="s")

  @functools.partial(
      pl.kernel,
      out_type=jax.ShapeDtypeStruct((NC, NPAD, D), jnp.float32),
      mesh=mesh,
      scratch_types=[
          pltpu.VMEM((2, 2, KG, BLK), jnp.int32),
          pltpu.VMEM((2, BLK, D), jnp.float32),
          pltpu.VMEM_SHARED((NPAD, D), jnp.float32),
          pltpu.SemaphoreType.DMA((2,)),
          pltpu.SemaphoreType.DMA((2,)),
          pltpu.SemaphoreType.DMA((2,)),
      ],
  )
  def k(g_hbm, idx_hbm, z_hbm, out_hbm, idx_v, buf, acc, isem, gsem, ssem):
    c = lax.axis_index("c")
    s = lax.axis_index("s")
    r0 = s * RPT
    pltpu.sync_copy(z_hbm, acc.at[pl.ds(r0, RPT)])

    def run(base, ng):
      pltpu.async_copy(idx_hbm.at[s, base], idx_v.at[0], isem.at[0])

      def body(grp, carry):
        pg = lax.rem(grp, 2)
        png = 1 - pg
        pltpu.make_async_copy(idx_hbm.at[s, base + grp], idx_v.at[pg],
                              isem.at[pg]).wait()

        @pl.when(grp < ng - 1)
        def _prefetch():
          pltpu.async_copy(idx_hbm.at[s, base + grp + 1], idx_v.at[png],
                           isem.at[png])

        gd = {}
        sd = {}
        gd[0] = pltpu.async_copy(g_hbm.at[idx_v.at[pg, 0, 0]], buf.at[0],
                                 gsem.at[0])
        gd[1] = pltpu.async_copy(g_hbm.at[idx_v.at[pg, 0, 1]], buf.at[1],
                                 gsem.at[1])
        for b in range(KG):
          bb = b % 2
          gd[b].wait()
          sd[b] = pltpu.async_copy(buf.at[bb], acc.at[idx_v.at[pg, 1, b]],
                                   ssem.at[bb], add=True)
          if b + 2 < KG:
            sd[b].wait()
            gd[b + 2] = pltpu.async_copy(g_hbm.at[idx_v.at[pg, 0, b + 2]],
                                         buf.at[bb], gsem.at[bb])
        sd[KG - 2].wait()
        sd[KG - 1].wait()
        return carry

      lax.fori_loop(0, ng, body, 0)

    @pl.when(c == 0)
    def _c0():
      run(0, NG0)

    @pl.when(c == 1)
    def _c1():
      run(NG0, NG1)

    plsc.subcore_barrier()
    pltpu.sync_copy(acc.at[pl.ds(r0, RPT)], out_hbm.at[c, pl.ds(r0, RPT)])

  return k(g, idx6, zbig)


def _tc_pre(x, W, degp):
  def body(x_ref, w_ref, dp_ref, dinv_ref, g_ref):
    deg = dp_ref[0, :N, 0:1] + dp_ref[1, :N, 0:1] + 1.0
    dinv = lax.rsqrt(deg)
    dinv_ref[...] = dinv
    h = jnp.dot(x_ref[...], w_ref[...], preferred_element_type=jnp.float32)
    g_ref[...] = h * dinv

  return pl.pallas_call(
      body,
      out_shape=(
          jax.ShapeDtypeStruct((N, 1), jnp.float32),
          jax.ShapeDtypeStruct((N, D), jnp.float32),
      ),
  )(x, W, degp)


def _tc_mid(S, g, dinv, b, Wn):
  def body(s_ref, g_ref, dinv_ref, b_ref, w_ref, gout_ref):
    t = s_ref[0, :N, :] + s_ref[1, :N, :] + g_ref[...]
    xn = jnp.maximum(t * dinv_ref[...] + b_ref[...], 0.0)
    h = jnp.dot(xn, w_ref[...], preferred_element_type=jnp.float32)
    gout_ref[...] = h * dinv_ref[...]

  return pl.pallas_call(
      body,
      out_shape=jax.ShapeDtypeStruct((N, D), jnp.float32),
  )(S, g, dinv, b.reshape(1, D), Wn)


def _tc_fin(S, g, dinv, b):
  def body(s_ref, g_ref, dinv_ref, b_ref, out_ref):
    t = s_ref[0, :N, :] + s_ref[1, :N, :] + g_ref[...]
    out_ref[...] = jnp.maximum(t * dinv_ref[...] + b_ref[...], 0.0)

  return pl.pallas_call(
      body,
      out_shape=jax.ShapeDtypeStruct((N, D), jnp.float32),
  )(S, g, dinv, b.reshape(1, D))


@jax.jit
def kernel(encoded_cfg_nodes, edge_index, W1, b1, W2, b2, W3, b3):
  x = encoded_cfg_nodes
  src = edge_index[0].astype(jnp.int32)
  dst = edge_index[1].astype(jnp.int32)
  e = src.shape[0]
  pad = EP - e
  srcp = jnp.concatenate([src, jnp.zeros((pad,), jnp.int32)]).reshape(
      NS, NGT, KG, BLK)
  dstp = jnp.concatenate([dst, jnp.full((pad,), N, jnp.int32)]).reshape(
      NS, NGT, KG, BLK)
  idx6 = jnp.stack([srcp, dstp], axis=2)
  dst4 = dstp.reshape(NC, NS, NBLK, BLK)
  ones128 = jnp.ones((BLK, D), jnp.float32)
  zbig = jnp.zeros((RPT, D), jnp.float32)

  degp = _sc_deg(dst4, ones128, zbig)
  dinv, g = _tc_pre(x, W1, degp)
  S = _sc_agg(g, idx6, zbig)
  g = _tc_mid(S, g, dinv, b1, W2)
  S = _sc_agg(g, idx6, zbig)
  g = _tc_mid(S, g, dinv, b2, W3)
  S = _sc_agg(g, idx6, zbig)
  return _tc_fin(S, g, dinv, b3)


# speedup vs baseline: 9.3632x; 1.0037x over previous
import functools

import jax
import jax.numpy as jnp
from jax import lax
from jax.experimental import pallas as pl
from jax.experimental.pallas import tpu as pltpu
from jax.experimental.pallas import tpu_sc as plsc

N = 10000
NPAD = 10016
D = 128
NC, NS = 2, 16
NBLK = 80
BLK = 128
EP = NC * NS * NBLK * BLK
TBLK = 160
NB0 = 120
NB1 = TBLK - NB0
RPT = 624
RPTL = NPAD - 15 * RPT


def _zero_my_rows(z_hbm, acc, s):
  @pl.when(s < 15)
  def _():
    pltpu.sync_copy(z_hbm.at[pl.ds(0, RPT)], acc.at[pl.ds(s * RPT, RPT)])

  @pl.when(s == 15)
  def _():
    pltpu.sync_copy(z_hbm, acc.at[pl.ds(15 * RPT, RPTL)])


def _copy_my_rows(acc, out_hbm, c, s):
  @pl.when(s < 15)
  def _():
    pltpu.sync_copy(acc.at[pl.ds(s * RPT, RPT)],
                    out_hbm.at[c, pl.ds(s * RPT, RPT)])

  @pl.when(s == 15)
  def _():
    pltpu.sync_copy(acc.at[pl.ds(15 * RPT, RPTL)],
                    out_hbm.at[c, pl.ds(15 * RPT, RPTL)])


def _sc_deg(dst4, ones16, zdeg):
  mesh = plsc.VectorSubcoreMesh(core_axis_name="c", subcore_axis_name="s")

  @functools.partial(
      pl.kernel,
      out_type=jax.ShapeDtypeStruct((NC, NPAD, D), jnp.float32),
      mesh=mesh,
      scratch_types=[
          pltpu.VMEM((NBLK, BLK), jnp.int32),
          pltpu.VMEM((BLK, D), jnp.float32),
          pltpu.VMEM_SHARED((NPAD, D), jnp.float32),
          pltpu.SemaphoreType.DMA((8,)),
      ],
  )
  def k(dst_hbm, ones_hbm, z_hbm, out_hbm, dst_v, ones_v, acc, ssem):
    c = lax.axis_index("c")
    s = lax.axis_index("s")
    _zero_my_rows(z_hbm, acc, s)
    pltpu.sync_copy(ones_hbm, ones_v)
    pltpu.sync_copy(dst_hbm.at[c, s], dst_v)
    plsc.subcore_barrier()

    def body(j, carry):
      sd = [
          pltpu.async_copy(ones_v, acc.at[dst_v.at[j * 8 + b]], ssem.at[b],
                           add=True) for b in range(8)
      ]
      for d in sd:
        d.wait()
      return carry

    lax.fori_loop(0, NBLK // 8, body, 0)
    plsc.subcore_barrier()
    _copy_my_rows(acc, out_hbm, c, s)

  return k(dst4, ones16, zdeg)


def _sc_agg(g, idx6, zbig):
  mesh = plsc.VectorSubcoreMesh(core_axis_name="c", subcore_axis_name="s")

  @functools.partial(
      pl.kernel,
      out_type=jax.ShapeDtypeStruct((NC, NPAD, D), jnp.float32),
      mesh=mesh,
      scratch_types=[
          pltpu.VMEM((4, 2, BLK), jnp.int32),
          pltpu.VMEM((3, BLK, D), jnp.float32),
          pltpu.VMEM_SHARED((NPAD, D), jnp.float32),
          pltpu.SemaphoreType.DMA((4,)),
          pltpu.SemaphoreType.DMA((3,)),
          pltpu.SemaphoreType.DMA((3,)),
      ],
  )
  def k(g_hbm, idx_hbm, z_hbm, out_hbm, idx_v, buf, acc, isem, gsem, ssem):
    c = lax.axis_index("c")
    s = lax.axis_index("s")
    _zero_my_rows(z_hbm, acc, s)
    plsc.subcore_barrier()

    def idx_load(j, slot):
      return pltpu.async_copy(idx_hbm.at[s, j], idx_v.at[slot], isem.at[slot])

    def idx_wait(j, slot):
      pltpu.make_async_copy(idx_hbm.at[s, j], idx_v.at[slot],
                            isem.at[slot]).wait()

    def gather(j4, b):
      return pltpu.async_copy(g_hbm.at[idx_v.at[j4, 0]], buf.at[b],
                              gsem.at[b])

    def gather_wait(j4, b):
      pltpu.make_async_copy(g_hbm.at[idx_v.at[j4, 0]], buf.at[b],
                            gsem.at[b]).wait()

    def scatter(j4, b):
      return pltpu.async_copy(buf.at[b], acc.at[idx_v.at[j4, 1]],
                              ssem.at[b], add=True)

    def scatter_wait(j4, b):
      pltpu.make_async_copy(buf.at[b], acc.at[idx_v.at[j4, 1]],
                            ssem.at[b]).wait()

    def run(base, nb):
      for jj in range(4):
        idx_load(base + jj, jj)
      idx_wait(base, 0)
      gather(0, 0)
      idx_wait(base + 1, 1)
      gather(1, 1)

      def body(j, carry):
        b = lax.rem(j, 3)
        bn = lax.rem(j + 2, 3)
        i0 = lax.rem(j, 4)
        i2 = lax.rem(j + 2, 4)
        i3 = lax.rem(j + 3, 4)
        gather_wait(i0, b)
        scatter(i0, b)

        @pl.when(j >= 1)
        def _():
          scatter_wait(i3, bn)

        @pl.when(jnp.logical_and(j >= 1, j + 3 < nb))
        def _():
          idx_load(base + j + 3, i3)

        @pl.when(j + 2 < nb)
        def _():
          idx_wait(base + j + 2, i2)
          gather(i2, bn)
        return carry

      lax.fori_loop(0, nb, body, 0)
      scatter_wait(lax.rem(nb - 1, 4), lax.rem(nb - 1, 3))

    @pl.when(c == 0)
    def _c0():
      run(0, NB0)

    @pl.when(c == 1)
    def _c1():
      run(NB0, NB1)

    plsc.subcore_barrier()
    _copy_my_rows(acc, out_hbm, c, s)

  return k(g, idx6, zbig)


def _tc_pre(x, W, degp):
  def body(x_ref, w_ref, dp_ref, dinv_ref, g_ref):
    deg = dp_ref[0, :N, 0:1] + dp_ref[1, :N, 0:1] + 1.0
    dinv = lax.rsqrt(deg)
    dinv_ref[...] = dinv
    h = jnp.dot(x_ref[...], w_ref[...], preferred_element_type=jnp.float32)
    g_ref[...] = h * dinv

  return pl.pallas_call(
      body,
      out_shape=(
          jax.ShapeDtypeStruct((N, 1), jnp.float32),
          jax.ShapeDtypeStruct((N, D), jnp.float32),
      ),
  )(x, W, degp)


def _tc_mid(S, g, dinv, b, Wn):
  def body(s_ref, g_ref, dinv_ref, b_ref, w_ref, gout_ref):
    t = s_ref[0, :N, :] + s_ref[1, :N, :] + g_ref[...]
    xn = jnp.maximum(t * dinv_ref[...] + b_ref[...], 0.0)
    h = jnp.dot(xn, w_ref[...], preferred_element_type=jnp.float32)
    gout_ref[...] = h * dinv_ref[...]

  return pl.pallas_call(
      body,
      out_shape=jax.ShapeDtypeStruct((N, D), jnp.float32),
  )(S, g, dinv, b.reshape(1, D), Wn)


def _tc_fin(S, g, dinv, b):
  def body(s_ref, g_ref, dinv_ref, b_ref, out_ref):
    t = s_ref[0, :N, :] + s_ref[1, :N, :] + g_ref[...]
    out_ref[...] = jnp.maximum(t * dinv_ref[...] + b_ref[...], 0.0)

  return pl.pallas_call(
      body,
      out_shape=jax.ShapeDtypeStruct((N, D), jnp.float32),
  )(S, g, dinv, b.reshape(1, D))


@jax.jit
def kernel(encoded_cfg_nodes, edge_index, W1, b1, W2, b2, W3, b3):
  x = encoded_cfg_nodes
  src = edge_index[0].astype(jnp.int32)
  dst = edge_index[1].astype(jnp.int32)
  e = src.shape[0]
  pad = EP - e
  srcp = jnp.concatenate([src, jnp.zeros((pad,), jnp.int32)]).reshape(
      NS, TBLK, BLK)
  dstp = jnp.concatenate([dst, jnp.full((pad,), N, jnp.int32)]).reshape(
      NS, TBLK, BLK)
  idx6 = jnp.stack([srcp, dstp], axis=2)
  dst4 = dstp.reshape(NC, NS, NBLK, BLK)
  ones128 = jnp.ones((BLK, D), jnp.float32)
  zbig = jnp.zeros((RPTL, D), jnp.float32)

  degp = _sc_deg(dst4, ones128, zbig)
  dinv, g = _tc_pre(x, W1, degp)
  S = _sc_agg(g, idx6, zbig)
  g = _tc_mid(S, g, dinv, b1, W2)
  S = _sc_agg(g, idx6, zbig)
  g = _tc_mid(S, g, dinv, b2, W3)
  S = _sc_agg(g, idx6, zbig)
  return _tc_fin(S, g, dinv, b3)
